# initial kernel scaffold (unmeasured)
import jax
import jax.numpy as jnp
from jax import lax
from jax.experimental import pallas as pl
from jax.experimental.pallas import tpu as pltpu

N_DEV = 16
M = 768
N_OUT = 768
CHUNK = M // N_DEV

_RS_MASKS = (8, 4, 2, 1)
_AG_MASKS = (1, 2, 4, 8)
_RS_OFFS = (0, 384, 576, 672)


def kernel(x, Wg, Wu, Wd):
    def body(x_ref, wg_ref, wu_ref, wd_ref, out_ref,
             acc_ref, rs_buf, rs_send, rs_recv, ag_send, ag_recv):
        me = lax.axis_index("i")

        barrier = pltpu.get_barrier_semaphore()
        for mask in _RS_MASKS:
            pl.semaphore_signal(
                barrier, inc=1, device_id=(me ^ mask,),
                device_id_type=pl.DeviceIdType.MESH,
            )
        pl.semaphore_wait(barrier, 4)

        xb = x_ref[...].astype(jnp.bfloat16)
        gate = jnp.dot(xb, wg_ref[...].astype(jnp.bfloat16),
                       preferred_element_type=jnp.float32)
        up = jnp.dot(xb, wu_ref[...].astype(jnp.bfloat16),
                     preferred_element_type=jnp.float32)
        h = (gate * (up * jax.nn.sigmoid(up))).astype(jnp.bfloat16)
        acc_ref[...] = jnp.dot(h, wd_ref[...].astype(jnp.bfloat16),
                               preferred_element_type=jnp.float32)

        lo = jnp.int32(0)
        size = M
        for k, mask in enumerate(_RS_MASKS):
            half = size // 2
            has_bit = (me & mask) != 0
            keep_lo = lo + jnp.where(has_bit, half, 0)
            send_lo = lo + jnp.where(has_bit, 0, half)
            rdma = pltpu.make_async_remote_copy(
                src_ref=acc_ref.at[pl.ds(send_lo, half)],
                dst_ref=rs_buf.at[pl.ds(_RS_OFFS[k], half)],
                send_sem=rs_send.at[k],
                recv_sem=rs_recv.at[k],
                device_id=(me ^ mask,),
                device_id_type=pl.DeviceIdType.MESH,
            )
            rdma.start()
            rdma.wait()
            cur = pl.load(acc_ref, (pl.ds(keep_lo, half), slice(None)))
            inc = pl.load(rs_buf, (pl.ds(_RS_OFFS[k], half), slice(None)))
            pl.store(acc_ref, (pl.ds(keep_lo, half), slice(None)), cur + inc)
            lo = keep_lo
            size = half

        pl.store(out_ref, (pl.ds(lo, size), slice(None)),
                 pl.load(acc_ref, (pl.ds(lo, size), slice(None))))

        for k, mask in enumerate(_AG_MASKS):
            rdma = pltpu.make_async_remote_copy(
                src_ref=out_ref.at[pl.ds(lo, size)],
                dst_ref=out_ref.at[pl.ds(lo, size)],
                send_sem=ag_send.at[k],
                recv_sem=ag_recv.at[k],
                device_id=(me ^ mask,),
                device_id_type=pl.DeviceIdType.MESH,
            )
            rdma.start()
            rdma.wait()
            lo = lo - (me & mask) * CHUNK
            size = size * 2

    return pl.pallas_call(
        body,
        out_shape=jax.ShapeDtypeStruct((M, N_OUT), jnp.float32),
        in_specs=[pl.BlockSpec(memory_space=pltpu.VMEM)] * 4,
        out_specs=pl.BlockSpec(memory_space=pltpu.VMEM),
        scratch_shapes=[
            pltpu.VMEM((M, N_OUT), jnp.float32),
            pltpu.VMEM((720, N_OUT), jnp.float32),
            pltpu.SemaphoreType.DMA((4,)),
            pltpu.SemaphoreType.DMA((4,)),
            pltpu.SemaphoreType.DMA((4,)),
            pltpu.SemaphoreType.DMA((4,)),
        ],
        compiler_params=pltpu.CompilerParams(collective_id=0),
    )(x, Wg, Wu, Wd)


# baseline (device time: 107701 ns/iter reference)
import jax
import jax.numpy as jnp
from jax import lax
from jax.experimental import pallas as pl
from jax.experimental.pallas import tpu as pltpu

N_DEV = 16
M = 768
N_OUT = 768
CHUNK = M // N_DEV

_RS_MASKS = (8, 4, 2, 1)
_AG_MASKS = (1, 2, 4, 8)
_RS_OFFS = (0, 384, 576, 672)


def kernel(x, Wg, Wu, Wd):
    def body(x_ref, wg_ref, wu_ref, wd_ref, out_ref,
             acc_ref, rs_buf, rs_send, rs_recv, ag_send, ag_recv):
        me = lax.axis_index("i")

        barrier = pltpu.get_barrier_semaphore()
        for mask in _RS_MASKS:
            pl.semaphore_signal(
                barrier, inc=1, device_id=(me ^ mask,),
                device_id_type=pl.DeviceIdType.MESH,
            )
        pl.semaphore_wait(barrier, 4)

        xb = x_ref[...].astype(jnp.bfloat16)
        gate = jnp.dot(xb, wg_ref[...].astype(jnp.bfloat16),
                       preferred_element_type=jnp.float32)
        up = jnp.dot(xb, wu_ref[...].astype(jnp.bfloat16),
                     preferred_element_type=jnp.float32)
        h = (gate * (up * jax.nn.sigmoid(up))).astype(jnp.bfloat16)
        acc_ref[...] = jnp.dot(h, wd_ref[...].astype(jnp.bfloat16),
                               preferred_element_type=jnp.float32)

        lo = jnp.int32(0)
        size = M
        for k, mask in enumerate(_RS_MASKS):
            half = size // 2
            has_bit = (me & mask) != 0
            keep_lo = lo + jnp.where(has_bit, half, 0)
            send_lo = lo + jnp.where(has_bit, 0, half)
            rdma = pltpu.make_async_remote_copy(
                src_ref=acc_ref.at[pl.ds(send_lo, half)],
                dst_ref=rs_buf.at[pl.ds(_RS_OFFS[k], half)],
                send_sem=rs_send.at[k],
                recv_sem=rs_recv.at[k],
                device_id=(me ^ mask,),
                device_id_type=pl.DeviceIdType.MESH,
            )
            rdma.start()
            rdma.wait()
            acc_ref[pl.ds(keep_lo, half), :] = (
                acc_ref[pl.ds(keep_lo, half), :]
                + rs_buf[pl.ds(_RS_OFFS[k], half), :]
            )
            lo = keep_lo
            size = half

        out_ref[pl.ds(lo, size), :] = acc_ref[pl.ds(lo, size), :]

        for k, mask in enumerate(_AG_MASKS):
            rdma = pltpu.make_async_remote_copy(
                src_ref=out_ref.at[pl.ds(lo, size)],
                dst_ref=out_ref.at[pl.ds(lo, size)],
                send_sem=ag_send.at[k],
                recv_sem=ag_recv.at[k],
                device_id=(me ^ mask,),
                device_id_type=pl.DeviceIdType.MESH,
            )
            rdma.start()
            rdma.wait()
            lo = lo - (me & mask) * CHUNK
            size = size * 2

    return pl.pallas_call(
        body,
        out_shape=jax.ShapeDtypeStruct((M, N_OUT), jnp.float32),
        in_specs=[pl.BlockSpec(memory_space=pltpu.VMEM)] * 4,
        out_specs=pl.BlockSpec(memory_space=pltpu.VMEM),
        scratch_shapes=[
            pltpu.VMEM((M, N_OUT), jnp.float32),
            pltpu.VMEM((720, N_OUT), jnp.float32),
            pltpu.SemaphoreType.DMA((4,)),
            pltpu.SemaphoreType.DMA((4,)),
            pltpu.SemaphoreType.DMA((4,)),
            pltpu.SemaphoreType.DMA((4,)),
        ],
        compiler_params=pltpu.CompilerParams(collective_id=0),
    )(x, Wg, Wu, Wd)


# device time: 67742 ns/iter; 1.5899x vs baseline; 1.5899x over previous
import jax
import jax.numpy as jnp
from jax import lax
from jax.experimental import pallas as pl
from jax.experimental.pallas import tpu as pltpu

N_DEV = 16
M = 768
N_OUT = 768
CHUNK = M // N_DEV

_RS_MASKS = (8, 4, 2, 1)
_AG_MASKS = (1, 2, 4, 8)
_RS_OFFS = (0, 384, 576, 672)


def kernel(x, Wg, Wu, Wd):
    def body(x_ref, wg_ref, wu_ref, wd_ref, out_ref,
             acc_ref, rs_stage, rs_buf, ag_buf,
             rs_send, rs_recv, ag_send, ag_recv):
        me = lax.axis_index("i")

        barrier = pltpu.get_barrier_semaphore()
        for mask in _RS_MASKS:
            pl.semaphore_signal(
                barrier, inc=1, device_id=(me ^ mask,),
                device_id_type=pl.DeviceIdType.MESH,
            )
        pl.semaphore_wait(barrier, 4)

        xb = x_ref[...].astype(jnp.bfloat16)
        gate = jnp.dot(xb, wg_ref[...].astype(jnp.bfloat16),
                       preferred_element_type=jnp.float32)
        up = jnp.dot(xb, wu_ref[...].astype(jnp.bfloat16),
                     preferred_element_type=jnp.float32)
        h = (gate * (up * jax.nn.sigmoid(up))).astype(jnp.bfloat16)
        acc_ref[...] = jnp.dot(h, wd_ref[...].astype(jnp.bfloat16),
                               preferred_element_type=jnp.float32)

        lo = jnp.int32(0)
        size = M
        rdmas = []
        for k, mask in enumerate(_RS_MASKS):
            half = size // 2
            has_bit = (me & mask) != 0
            keep_lo = lo + jnp.where(has_bit, half, 0)
            send_lo = lo + jnp.where(has_bit, 0, half)
            rs_stage[pl.ds(_RS_OFFS[k], half), :] = (
                acc_ref[pl.ds(send_lo, half), :].astype(jnp.bfloat16)
            )
            rdma = pltpu.make_async_remote_copy(
                src_ref=rs_stage.at[pl.ds(_RS_OFFS[k], half)],
                dst_ref=rs_buf.at[pl.ds(_RS_OFFS[k], half)],
                send_sem=rs_send.at[k],
                recv_sem=rs_recv.at[k],
                device_id=(me ^ mask,),
                device_id_type=pl.DeviceIdType.MESH,
            )
            rdma.start()
            rdma.wait_recv()
            rdmas.append(rdma)
            acc_ref[pl.ds(keep_lo, half), :] = (
                acc_ref[pl.ds(keep_lo, half), :]
                + rs_buf[pl.ds(_RS_OFFS[k], half), :].astype(jnp.float32)
            )
            lo = keep_lo
            size = half

        ag_buf[pl.ds(lo, size), :] = (
            acc_ref[pl.ds(lo, size), :].astype(jnp.bfloat16)
        )

        for k, mask in enumerate(_AG_MASKS):
            rdma = pltpu.make_async_remote_copy(
                src_ref=ag_buf.at[pl.ds(lo, size)],
                dst_ref=ag_buf.at[pl.ds(lo, size)],
                send_sem=ag_send.at[k],
                recv_sem=ag_recv.at[k],
                device_id=(me ^ mask,),
                device_id_type=pl.DeviceIdType.MESH,
            )
            rdma.start()
            rdma.wait_recv()
            rdmas.append(rdma)
            lo = lo - (me & mask) * CHUNK
            size = size * 2

        out_ref[...] = ag_buf[...].astype(jnp.float32)

        for rdma in rdmas:
            rdma.wait_send()

    return pl.pallas_call(
        body,
        out_shape=jax.ShapeDtypeStruct((M, N_OUT), jnp.float32),
        in_specs=[pl.BlockSpec(memory_space=pltpu.VMEM)] * 4,
        out_specs=pl.BlockSpec(memory_space=pltpu.VMEM),
        scratch_shapes=[
            pltpu.VMEM((M, N_OUT), jnp.float32),
            pltpu.VMEM((720, N_OUT), jnp.bfloat16),
            pltpu.VMEM((720, N_OUT), jnp.bfloat16),
            pltpu.VMEM((M, N_OUT), jnp.bfloat16),
            pltpu.SemaphoreType.DMA((4,)),
            pltpu.SemaphoreType.DMA((4,)),
            pltpu.SemaphoreType.DMA((4,)),
            pltpu.SemaphoreType.DMA((4,)),
        ],
        compiler_params=pltpu.CompilerParams(collective_id=0),
    )(x, Wg, Wu, Wd)


# device time: 59997 ns/iter; 1.7951x vs baseline; 1.1291x over previous
import jax
import jax.numpy as jnp
from jax import lax
from jax.experimental import pallas as pl
from jax.experimental.pallas import tpu as pltpu

N_DEV = 16
M = 768
N_OUT = 768
CHUNK = M // N_DEV

_RS_MASKS = (1, 2, 4, 8)
_AG_MASKS = (8, 4, 2, 1)
_RS_OFFS = (0, 384, 576, 672)


def kernel(x, Wg, Wu, Wd):
    def body(x_ref, wg_ref, wu_ref, wd_ref, out_ref,
             acc_ref, rs_stage, rs_buf, ag_buf,
             rs_send, rs_recv, ag_send, ag_recv):
        me = lax.axis_index("i")

        barrier = pltpu.get_barrier_semaphore()
        for mask in _RS_MASKS:
            pl.semaphore_signal(
                barrier, inc=1, device_id=(me ^ mask,),
                device_id_type=pl.DeviceIdType.MESH,
            )
        pl.semaphore_wait(barrier, 4)

        xb = x_ref[...].astype(jnp.bfloat16)
        gate = jnp.dot(xb, wg_ref[...].astype(jnp.bfloat16),
                       preferred_element_type=jnp.float32)
        up = jnp.dot(xb, wu_ref[...].astype(jnp.bfloat16),
                     preferred_element_type=jnp.float32)
        h = (gate * (up * jax.nn.sigmoid(up))).astype(jnp.bfloat16)
        acc_ref[...] = jnp.dot(h, wd_ref[...].astype(jnp.bfloat16),
                               preferred_element_type=jnp.float32)

        lo = jnp.int32(0)
        size = M
        rdmas = []
        for k, mask in enumerate(_RS_MASKS):
            half = size // 2
            has_bit = (me & mask) != 0
            keep_lo = lo + jnp.where(has_bit, half, 0)
            send_lo = lo + jnp.where(has_bit, 0, half)
            rs_stage[pl.ds(_RS_OFFS[k], half), :] = (
                acc_ref[pl.ds(send_lo, half), :].astype(jnp.bfloat16)
            )
            rdma = pltpu.make_async_remote_copy(
                src_ref=rs_stage.at[pl.ds(_RS_OFFS[k], half)],
                dst_ref=rs_buf.at[pl.ds(_RS_OFFS[k], half)],
                send_sem=rs_send.at[k],
                recv_sem=rs_recv.at[k],
                device_id=(me ^ mask,),
                device_id_type=pl.DeviceIdType.MESH,
            )
            rdma.start()
            rdma.wait_recv()
            rdmas.append(rdma)
            acc_ref[pl.ds(keep_lo, half), :] = (
                acc_ref[pl.ds(keep_lo, half), :]
                + rs_buf[pl.ds(_RS_OFFS[k], half), :].astype(jnp.float32)
            )
            lo = keep_lo
            size = half

        ag_buf[pl.ds(lo, size), :] = (
            acc_ref[pl.ds(lo, size), :].astype(jnp.bfloat16)
        )

        for k, mask in enumerate(_AG_MASKS):
            rdma = pltpu.make_async_remote_copy(
                src_ref=ag_buf.at[pl.ds(lo, size)],
                dst_ref=ag_buf.at[pl.ds(lo, size)],
                send_sem=ag_send.at[k],
                recv_sem=ag_recv.at[k],
                device_id=(me ^ mask,),
                device_id_type=pl.DeviceIdType.MESH,
            )
            rdma.start()
            rdma.wait_recv()
            rdmas.append(rdma)
            lo = lo - jnp.where((me & mask) != 0, size, 0)
            size = size * 2

        out_ref[...] = ag_buf[...].astype(jnp.float32)

        for rdma in rdmas:
            rdma.wait_send()

    return pl.pallas_call(
        body,
        out_shape=jax.ShapeDtypeStruct((M, N_OUT), jnp.float32),
        in_specs=[pl.BlockSpec(memory_space=pltpu.VMEM)] * 4,
        out_specs=pl.BlockSpec(memory_space=pltpu.VMEM),
        scratch_shapes=[
            pltpu.VMEM((M, N_OUT), jnp.float32),
            pltpu.VMEM((720, N_OUT), jnp.bfloat16),
            pltpu.VMEM((720, N_OUT), jnp.bfloat16),
            pltpu.VMEM((M, N_OUT), jnp.bfloat16),
            pltpu.SemaphoreType.DMA((4,)),
            pltpu.SemaphoreType.DMA((4,)),
            pltpu.SemaphoreType.DMA((4,)),
            pltpu.SemaphoreType.DMA((4,)),
        ],
        compiler_params=pltpu.CompilerParams(collective_id=0),
    )(x, Wg, Wu, Wd)


# device time: 47275 ns/iter; 2.2782x vs baseline; 1.2691x over previous
import jax
import jax.numpy as jnp
from jax import lax
from jax.experimental import pallas as pl
from jax.experimental.pallas import tpu as pltpu

N_DEV = 16
M = 768
N_OUT = 768
CHUNK = M // N_DEV


def kernel(x, Wg, Wu, Wd):
    def body(x_ref, wg_ref, wu_ref, wd_ref, out_ref,
             acc_ref, stage, rs_buf, ag_buf,
             rs_send, rs_recv, ag_send, ag_recv):
        me = lax.axis_index("i")

        barrier = pltpu.get_barrier_semaphore()
        for j in range(1, N_DEV):
            pl.semaphore_signal(
                barrier, inc=1, device_id=((me + j) % N_DEV,),
                device_id_type=pl.DeviceIdType.MESH,
            )
        pl.semaphore_wait(barrier, N_DEV - 1)

        xb = x_ref[...].astype(jnp.bfloat16)
        gate = jnp.dot(xb, wg_ref[...].astype(jnp.bfloat16),
                       preferred_element_type=jnp.float32)
        up = jnp.dot(xb, wu_ref[...].astype(jnp.bfloat16),
                     preferred_element_type=jnp.float32)
        h = (gate * (up * jax.nn.sigmoid(up))).astype(jnp.bfloat16)
        acc_ref[...] = jnp.dot(h, wd_ref[...].astype(jnp.bfloat16),
                               preferred_element_type=jnp.float32)
        stage[...] = acc_ref[...].astype(jnp.bfloat16)

        rs_rdmas = []
        for j in range(1, N_DEV):
            t = (me + j) % N_DEV
            rdma = pltpu.make_async_remote_copy(
                src_ref=stage.at[pl.ds(t * CHUNK, CHUNK)],
                dst_ref=rs_buf.at[j - 1],
                send_sem=rs_send.at[j - 1],
                recv_sem=rs_recv.at[j - 1],
                device_id=(t,),
                device_id_type=pl.DeviceIdType.MESH,
            )
            rdma.start()
            rs_rdmas.append(rdma)
        for rdma in rs_rdmas:
            rdma.wait_recv()

        my_lo = me * CHUNK
        reduced = acc_ref[pl.ds(my_lo, CHUNK), :] + jnp.sum(
            rs_buf[...].astype(jnp.float32), axis=0
        )
        ag_buf[pl.ds(my_lo, CHUNK), :] = reduced.astype(jnp.bfloat16)

        ag_rdmas = []
        for j in range(1, N_DEV):
            t = (me + j) % N_DEV
            rdma = pltpu.make_async_remote_copy(
                src_ref=ag_buf.at[pl.ds(my_lo, CHUNK)],
                dst_ref=ag_buf.at[pl.ds(my_lo, CHUNK)],
                send_sem=ag_send.at[j - 1],
                recv_sem=ag_recv.at[j - 1],
                device_id=(t,),
                device_id_type=pl.DeviceIdType.MESH,
            )
            rdma.start()
            ag_rdmas.append(rdma)
        for rdma in ag_rdmas:
            rdma.wait_recv()

        out_ref[...] = ag_buf[...].astype(jnp.float32)

        for rdma in rs_rdmas + ag_rdmas:
            rdma.wait_send()

    return pl.pallas_call(
        body,
        out_shape=jax.ShapeDtypeStruct((M, N_OUT), jnp.float32),
        in_specs=[pl.BlockSpec(memory_space=pltpu.VMEM)] * 4,
        out_specs=pl.BlockSpec(memory_space=pltpu.VMEM),
        scratch_shapes=[
            pltpu.VMEM((M, N_OUT), jnp.float32),
            pltpu.VMEM((M, N_OUT), jnp.bfloat16),
            pltpu.VMEM((N_DEV - 1, CHUNK, N_OUT), jnp.bfloat16),
            pltpu.VMEM((M, N_OUT), jnp.bfloat16),
            pltpu.SemaphoreType.DMA((N_DEV - 1,)),
            pltpu.SemaphoreType.DMA((N_DEV - 1,)),
            pltpu.SemaphoreType.DMA((N_DEV - 1,)),
            pltpu.SemaphoreType.DMA((N_DEV - 1,)),
        ],
        compiler_params=pltpu.CompilerParams(collective_id=0),
    )(x, Wg, Wu, Wd)
